# baseline (speedup 1.0000x reference)
import functools

import jax
import jax.numpy as jnp
from jax import lax
from jax.experimental import pallas as pl
from jax.experimental.pallas import tpu as pltpu
from jax.experimental.pallas import tpu_sc as plsc

N = 10000
D = 128
HD = D // 2
NPAD = 10240
NC = 2
NS = 16
NW = NC * NS
CHUNK = 128
NBUF = 2
ROWS_PER_TILE = NPAD // NS


def _sc_degree(dst_w, n_chunks):
    mesh = plsc.VectorSubcoreMesh(core_axis_name="c", subcore_axis_name="s")

    @functools.partial(
        pl.kernel,
        out_type=jax.ShapeDtypeStruct((NC, NPAD), jnp.float32),
        mesh=mesh,
        scratch_types=[
            pltpu.VMEM((n_chunks, CHUNK), jnp.int32),
            pltpu.VMEM((CHUNK,), jnp.float32),
            pltpu.VMEM((ROWS_PER_TILE,), jnp.float32),
            pltpu.VMEM_SHARED((NPAD,), jnp.float32),
        ],
    )
    def deg_kernel(dst_hbm, out_hbm, didx, ones_v, zbuf, acc):
        cid = lax.axis_index("c")
        sid = lax.axis_index("s")
        wid = sid * NC + cid

        def fill(i, _):
            ones_v[pl.ds(i * 16, 16)] = jnp.ones((16,), jnp.float32)
            return 0
        lax.fori_loop(0, CHUNK // 16, fill, 0)

        def zfill(i, _):
            zbuf[pl.ds(i * 16, 16)] = jnp.zeros((16,), jnp.float32)
            return 0
        lax.fori_loop(0, ROWS_PER_TILE // 16, zfill, 0)

        base = sid * ROWS_PER_TILE
        pltpu.sync_copy(zbuf, acc.at[pl.ds(base, ROWS_PER_TILE)])
        plsc.subcore_barrier()

        pltpu.sync_copy(dst_hbm.at[wid], didx)

        def step(j, _):
            pltpu.sync_copy(ones_v, acc.at[didx.at[j]], add=True)
            return 0
        lax.fori_loop(0, n_chunks, step, 0)
        plsc.subcore_barrier()

        pltpu.sync_copy(acc.at[pl.ds(base, ROWS_PER_TILE)],
                        out_hbm.at[cid, pl.ds(base, ROWS_PER_TILE)])

    return deg_kernel(dst_w)


def _sc_segsum(hs, src_w, dst_w, n_chunks):
    mesh = plsc.VectorSubcoreMesh(core_axis_name="c", subcore_axis_name="s")

    @functools.partial(
        pl.kernel,
        out_type=jax.ShapeDtypeStruct((NC, NPAD, D), jnp.float32),
        mesh=mesh,
        scratch_types=[
            pltpu.VMEM((n_chunks // 2, CHUNK), jnp.int32),
            pltpu.VMEM((n_chunks // 2, CHUNK), jnp.int32),
            pltpu.VMEM((NBUF, CHUNK, D), jnp.float32),
            pltpu.SemaphoreType.DMA((NBUF,)),
            pltpu.SemaphoreType.DMA((NBUF,)),
            pltpu.VMEM_SHARED((NPAD, D), jnp.float32),
        ],
    )
    def seg_kernel(hs_hbm, src_hbm, dst_hbm, out_hbm, sidx, didx,
                   rows, sem_g, sem_s, acc):
        cid = lax.axis_index("c")
        sid = lax.axis_index("s")
        wid = sid * NC + cid
        nh = n_chunks // 2

        def zr(r, _):
            def zc(c, _):
                rows[0, r, pl.ds(c * 16, 16)] = jnp.zeros((16,), jnp.float32)
                return 0
            return lax.fori_loop(0, D // 16, zc, 0)
        lax.fori_loop(0, CHUNK, zr, 0)

        base = sid * ROWS_PER_TILE
        for k in range(ROWS_PER_TILE // CHUNK):
            pltpu.sync_copy(rows.at[0], acc.at[pl.ds(base + k * CHUNK, CHUNK)])
        plsc.subcore_barrier()

        def g_start(j, b):
            pltpu.async_copy(hs_hbm.at[sidx.at[j]], rows.at[b], sem_g.at[b])

        def g_wait(j, b):
            pltpu.make_async_copy(hs_hbm.at[sidx.at[j]], rows.at[b],
                                  sem_g.at[b]).wait()

        def s_start(j, b):
            pltpu.async_copy(rows.at[b], acc.at[didx.at[j]], sem_s.at[b],
                             add=True)

        def s_wait(j, b):
            pltpu.make_async_copy(rows.at[b], acc.at[didx.at[j]],
                                  sem_s.at[b]).wait()

        for ph in range(2):
            pltpu.sync_copy(src_hbm.at[wid, pl.ds(ph * nh, nh)], sidx)
            pltpu.sync_copy(dst_hbm.at[wid, pl.ds(ph * nh, nh)], didx)

            n_groups = nh // NBUF
            for b in range(NBUF):
                g_start(b, b)

            def group(g, _):
                j0 = g * NBUF
                for b in range(NBUF):
                    g_wait(j0 + b, b)
                    s_start(j0 + b, b)
                for b in range(NBUF):
                    s_wait(j0 + b, b)
                    g_start(j0 + NBUF + b, b)
                return 0
            lax.fori_loop(0, n_groups - 1, group, 0)

            j0 = (n_groups - 1) * NBUF
            for b in range(NBUF):
                g_wait(j0 + b, b)
                s_start(j0 + b, b)
            for b in range(NBUF):
                s_wait(j0 + b, b)

        plsc.subcore_barrier()
        pltpu.sync_copy(acc.at[pl.ds(base, ROWS_PER_TILE)],
                        out_hbm.at[cid, pl.ds(base, ROWS_PER_TILE)])

    return seg_kernel(hs, src_w, dst_w)


_BLK = 1280
_GRID = NPAD // _BLK


def _tc1_body(x_ref, w_ref, deg_ref, hs_ref, d_ref):
    deg = deg_ref[0, :] + deg_ref[1, :] + 1.0
    dv = lax.rsqrt(deg)
    d_ref[pl.program_id(0), :] = dv
    g = jnp.dot(x_ref[...], w_ref[...], preferred_element_type=jnp.float32)
    hs_ref[...] = g * dv[:, None]


def _tc2_body(s_ref, hs_ref, d_ref, b_ref, w_ref, out_ref):
    dv = d_ref[pl.program_id(0), :]
    s = s_ref[0] + s_ref[1] + hs_ref[...]
    h1 = jnp.maximum(dv[:, None] * s + b_ref[...][None, :], 0.0)
    g = jnp.dot(h1, w_ref[...], preferred_element_type=jnp.float32)
    out_ref[...] = g * dv[:, None]


def _tc3_body(s_ref, hs_ref, d_ref, b_ref, w_ref, bl_ref, out_ref):
    dv = d_ref[pl.program_id(0), :]
    s = s_ref[0] + s_ref[1] + hs_ref[...]
    h2 = dv[:, None] * s + b_ref[...][None, :]
    out_ref[...] = (jnp.dot(h2, w_ref[...], preferred_element_type=jnp.float32)
                    + bl_ref[...][None, :])


def kernel(x, edge_index, W1, b1, W2, b2, Wl, bl):
    E = edge_index.shape[1]
    n_chunks = -(-E // (NW * CHUNK * NBUF)) * NBUF
    epad = n_chunks * NW * CHUNK - E

    ei = edge_index.astype(jnp.int32)
    src_p = jnp.concatenate([ei[0], jnp.zeros((epad,), jnp.int32)])
    dst_p = jnp.concatenate([ei[1], jnp.full((epad,), N, jnp.int32)])
    src_w = src_p.reshape(NW, n_chunks, CHUNK)
    dst_w = dst_p.reshape(NW, n_chunks, CHUNK)
    x_pad = jnp.pad(x, ((0, NPAD - N), (0, 0)))

    deg_p = _sc_degree(dst_w, n_chunks)

    row = lambda i: (i, 0)
    full = lambda i: (0, 0)
    dspec = pl.BlockSpec((_GRID, _BLK), lambda i: (0, 0))
    sspec = pl.BlockSpec((NC, _BLK, D), lambda i: (0, i, 0))
    bspec = pl.BlockSpec((D,), lambda i: (0,))

    hs1, d = pl.pallas_call(
        _tc1_body,
        grid=(_GRID,),
        in_specs=[
            pl.BlockSpec((_BLK, D), row),
            pl.BlockSpec((D, D), full),
            pl.BlockSpec((NC, _BLK), lambda i: (0, i)),
        ],
        out_specs=[pl.BlockSpec((_BLK, D), row), dspec],
        out_shape=[jax.ShapeDtypeStruct((NPAD, D), jnp.float32),
                   jax.ShapeDtypeStruct((_GRID, _BLK), jnp.float32)],
    )(x_pad, W1, deg_p)

    s1 = _sc_segsum(hs1, src_w, dst_w, n_chunks)

    hs2 = pl.pallas_call(
        _tc2_body,
        grid=(_GRID,),
        in_specs=[sspec, pl.BlockSpec((_BLK, D), row), dspec, bspec,
                  pl.BlockSpec((D, D), full)],
        out_specs=pl.BlockSpec((_BLK, D), row),
        out_shape=jax.ShapeDtypeStruct((NPAD, D), jnp.float32),
        input_output_aliases={1: 0},
    )(s1, hs1, d, b1, W2)

    s2 = _sc_segsum(hs2, src_w, dst_w, n_chunks)

    out = pl.pallas_call(
        _tc3_body,
        grid=(_GRID,),
        in_specs=[sspec, pl.BlockSpec((_BLK, D), row), dspec, bspec,
                  pl.BlockSpec((D, D), full), bspec],
        out_specs=pl.BlockSpec((_BLK, D), row),
        out_shape=jax.ShapeDtypeStruct((NPAD, D), jnp.float32),
    )(s2, hs2, d, b2, Wl, bl)

    return (out[:N],)


# speedup vs baseline: 1.4113x; 1.4113x over previous
import functools

import jax
import jax.numpy as jnp
from jax import lax
from jax.experimental import pallas as pl
from jax.experimental.pallas import tpu as pltpu
from jax.experimental.pallas import tpu_sc as plsc

N = 10000
D = 128
HD = D // 2
NPAD = 10240
NC = 2
NS = 16
NW = NC * NS
CHUNK = 128
NBUF = 8
ROWS_PER_TILE = NPAD // NS


def _sc_degree(dst_w, n_chunks):
    mesh = plsc.VectorSubcoreMesh(core_axis_name="c", subcore_axis_name="s")

    @functools.partial(
        pl.kernel,
        out_type=jax.ShapeDtypeStruct((NC, NPAD), jnp.float32),
        mesh=mesh,
        scratch_types=[
            pltpu.VMEM((n_chunks, CHUNK), jnp.int32),
            pltpu.VMEM((CHUNK,), jnp.float32),
            pltpu.VMEM((ROWS_PER_TILE,), jnp.float32),
            pltpu.VMEM_SHARED((NPAD,), jnp.float32),
        ],
    )
    def deg_kernel(dst_hbm, out_hbm, didx, ones_v, zbuf, acc):
        cid = lax.axis_index("c")
        sid = lax.axis_index("s")
        wid = sid * NC + cid

        def fill(i, _):
            ones_v[pl.ds(i * 16, 16)] = jnp.ones((16,), jnp.float32)
            return 0
        lax.fori_loop(0, CHUNK // 16, fill, 0)

        def zfill(i, _):
            zbuf[pl.ds(i * 16, 16)] = jnp.zeros((16,), jnp.float32)
            return 0
        lax.fori_loop(0, ROWS_PER_TILE // 16, zfill, 0)

        base = sid * ROWS_PER_TILE
        pltpu.sync_copy(zbuf, acc.at[pl.ds(base, ROWS_PER_TILE)])
        plsc.subcore_barrier()

        pltpu.sync_copy(dst_hbm.at[wid], didx)

        def step(j, _):
            pltpu.sync_copy(ones_v, acc.at[didx.at[j]], add=True)
            return 0
        lax.fori_loop(0, n_chunks, step, 0)
        plsc.subcore_barrier()

        pltpu.sync_copy(acc.at[pl.ds(base, ROWS_PER_TILE)],
                        out_hbm.at[cid, pl.ds(base, ROWS_PER_TILE)])

    return deg_kernel(dst_w)


def _sc_segsum(hs_a, hs_b, src_w, dst_w, n_chunks):
    mesh = plsc.VectorSubcoreMesh(core_axis_name="c", subcore_axis_name="s")

    @functools.partial(
        pl.kernel,
        out_type=jax.ShapeDtypeStruct((NC, NPAD, HD), jnp.float32),
        mesh=mesh,
        scratch_types=[
            pltpu.VMEM((n_chunks // 2, CHUNK), jnp.int32),
            pltpu.VMEM((n_chunks // 2, CHUNK), jnp.int32),
            pltpu.VMEM((NBUF, CHUNK, HD), jnp.float32),
            pltpu.SemaphoreType.DMA((NBUF,)),
            pltpu.SemaphoreType.DMA((NBUF,)),
            pltpu.VMEM_SHARED((NPAD, HD), jnp.float32),
        ],
        compiler_params=pltpu.CompilerParams(use_tc_tiling_on_sc=False),
    )
    def seg_kernel(ha_hbm, hb_hbm, src_hbm, dst_hbm, out_hbm, sidx, didx,
                   rows, sem_g, sem_s, acc):
        cid = lax.axis_index("c")
        sid = lax.axis_index("s")

        def zr(r, _):
            def zc(c, _):
                rows[0, r, pl.ds(c * 16, 16)] = jnp.zeros((16,), jnp.float32)
                return 0
            return lax.fori_loop(0, HD // 16, zc, 0)
        lax.fori_loop(0, CHUNK, zr, 0)

        base = sid * ROWS_PER_TILE
        for k in range(ROWS_PER_TILE // CHUNK):
            pltpu.sync_copy(rows.at[0], acc.at[pl.ds(base + k * CHUNK, CHUNK)])
        plsc.subcore_barrier()

        nh = n_chunks // 2

        def run(tab):
            def g_start(j, b):
                pltpu.async_copy(tab.at[sidx.at[j]], rows.at[b], sem_g.at[b])

            def g_wait(j, b):
                pltpu.make_async_copy(tab.at[sidx.at[j]], rows.at[b],
                                      sem_g.at[b]).wait()

            def s_start(j, b):
                pltpu.async_copy(rows.at[b], acc.at[didx.at[j]], sem_s.at[b],
                                 add=True)

            def s_wait(j, b):
                pltpu.make_async_copy(rows.at[b], acc.at[didx.at[j]],
                                      sem_s.at[b]).wait()

            for ph in range(2):
                pltpu.sync_copy(src_hbm.at[sid, pl.ds(ph * nh, nh)], sidx)
                pltpu.sync_copy(dst_hbm.at[sid, pl.ds(ph * nh, nh)], didx)

                n_groups = nh // NBUF
                for b in range(NBUF):
                    g_start(b, b)

                def group(g, _):
                    j0 = g * NBUF
                    for b in range(NBUF):
                        g_wait(j0 + b, b)
                        s_start(j0 + b, b)
                    for b in range(NBUF):
                        s_wait(j0 + b, b)
                        g_start(j0 + NBUF + b, b)
                    return 0
                lax.fori_loop(0, n_groups - 1, group, 0)

                j0 = (n_groups - 1) * NBUF
                for b in range(NBUF):
                    g_wait(j0 + b, b)
                    s_start(j0 + b, b)
                for b in range(NBUF):
                    s_wait(j0 + b, b)

        @pl.when(cid == 0)
        def _():
            run(ha_hbm)

        @pl.when(cid == 1)
        def _():
            run(hb_hbm)

        plsc.subcore_barrier()
        pltpu.sync_copy(acc.at[pl.ds(base, ROWS_PER_TILE)],
                        out_hbm.at[cid, pl.ds(base, ROWS_PER_TILE)])

    return seg_kernel(hs_a, hs_b, src_w, dst_w)


_BLK = 1280
_GRID = NPAD // _BLK


def _tc1_body(x_ref, w_ref, deg_ref, hsa_ref, hsb_ref, d_ref):
    deg = deg_ref[0, :] + deg_ref[1, :] + 1.0
    dv = lax.rsqrt(deg)
    d_ref[pl.program_id(0), :] = dv
    g = jnp.dot(x_ref[...], w_ref[...], preferred_element_type=jnp.float32)
    hs = g * dv[:, None]
    hsa_ref[...] = hs[:, :HD]
    hsb_ref[...] = hs[:, HD:]


def _tc2_body(s_ref, hsa_ref, hsb_ref, d_ref, b_ref, w_ref,
              outa_ref, outb_ref):
    dv = d_ref[pl.program_id(0), :]
    s = jnp.concatenate([s_ref[0] + hsa_ref[...], s_ref[1] + hsb_ref[...]],
                        axis=1)
    h1 = jnp.maximum(dv[:, None] * s + b_ref[...][None, :], 0.0)
    g = jnp.dot(h1, w_ref[...], preferred_element_type=jnp.float32)
    hs = g * dv[:, None]
    outa_ref[...] = hs[:, :HD]
    outb_ref[...] = hs[:, HD:]


def _tc3_body(s_ref, hsa_ref, hsb_ref, d_ref, b_ref, w_ref, bl_ref, out_ref):
    dv = d_ref[pl.program_id(0), :]
    s = jnp.concatenate([s_ref[0] + hsa_ref[...], s_ref[1] + hsb_ref[...]],
                        axis=1)
    h2 = dv[:, None] * s + b_ref[...][None, :]
    out_ref[...] = (jnp.dot(h2, w_ref[...], preferred_element_type=jnp.float32)
                    + bl_ref[...][None, :])


def kernel(x, edge_index, W1, b1, W2, b2, Wl, bl):
    E = edge_index.shape[1]
    n_chunks = -(-E // (NS * CHUNK * NBUF)) * NBUF
    epad = n_chunks * NS * CHUNK - E
    n_chunks_deg = n_chunks * NS // NW

    ei = edge_index.astype(jnp.int32)
    src_p = jnp.concatenate([ei[0], jnp.zeros((epad,), jnp.int32)])
    dst_p = jnp.concatenate([ei[1], jnp.full((epad,), N, jnp.int32)])
    src_w = src_p.reshape(NS, n_chunks, CHUNK)
    dst_w = dst_p.reshape(NS, n_chunks, CHUNK)
    dst_deg = dst_p.reshape(NW, n_chunks_deg, CHUNK)
    x_pad = jnp.pad(x, ((0, NPAD - N), (0, 0)))

    deg_p = _sc_degree(dst_deg, n_chunks_deg)

    row = lambda i: (i, 0)
    full = lambda i: (0, 0)
    half = pl.BlockSpec((_BLK, HD), row)
    dspec = pl.BlockSpec((_GRID, _BLK), lambda i: (0, 0))
    sspec = pl.BlockSpec((NC, _BLK, HD), lambda i: (0, i, 0))
    bspec = pl.BlockSpec((D,), lambda i: (0,))
    half_out = jax.ShapeDtypeStruct((NPAD, HD), jnp.float32)

    hs1a, hs1b, d = pl.pallas_call(
        _tc1_body,
        grid=(_GRID,),
        in_specs=[
            pl.BlockSpec((_BLK, D), row),
            pl.BlockSpec((D, D), full),
            pl.BlockSpec((NC, _BLK), lambda i: (0, i)),
        ],
        out_specs=[half, half, dspec],
        out_shape=[half_out, half_out,
                   jax.ShapeDtypeStruct((_GRID, _BLK), jnp.float32)],
    )(x_pad, W1, deg_p)

    s1 = _sc_segsum(hs1a, hs1b, src_w, dst_w, n_chunks)

    hs2a, hs2b = pl.pallas_call(
        _tc2_body,
        grid=(_GRID,),
        in_specs=[sspec, half, half, dspec, bspec, pl.BlockSpec((D, D), full)],
        out_specs=[half, half],
        out_shape=[half_out, half_out],
        input_output_aliases={1: 0, 2: 1},
    )(s1, hs1a, hs1b, d, b1, W2)

    s2 = _sc_segsum(hs2a, hs2b, src_w, dst_w, n_chunks)

    out = pl.pallas_call(
        _tc3_body,
        grid=(_GRID,),
        in_specs=[sspec, half, half, dspec, bspec,
                  pl.BlockSpec((D, D), full), bspec],
        out_specs=pl.BlockSpec((_BLK, D), row),
        out_shape=jax.ShapeDtypeStruct((NPAD, D), jnp.float32),
    )(s2, hs2a, hs2b, d, b2, Wl, bl)

    return (out[:N],)
